# Initial kernel scaffold; baseline (speedup 1.0000x reference)
import functools

import jax
import jax.numpy as jnp
from jax import lax
from jax.experimental import pallas as pl
from jax.experimental.pallas import tpu as pltpu
from jax.experimental.pallas import tpu_sc as plsc

ROWS = 128
COLS = 32768
K = 64
L = 16
NVREG = COLS // L
BSHIFT = 21
NBUCKET = 1 << (31 - BSHIFT)
CAND_CAP = 4096
SIGN_MASK = 0x7FFFFFFF
NC = 2
NS = 16
ROWS_PER_W = ROWS // (NC * NS)


def _sc_body(in_hbm, out_hbm, row_v, hist_v, cand_v):
    wid = lax.axis_index("s") * NC + lax.axis_index("c")
    lane = lax.iota(jnp.int32, L)
    ones = jnp.ones((L,), jnp.int32)
    zeros = jnp.zeros((L,), jnp.int32)

    def per_row(r, carry):
        row = wid * ROWS_PER_W + r
        pltpu.sync_copy(in_hbm.at[row], row_v)

        def zero_loop(i, c):
            hist_v[pl.ds(i * L, L)] = zeros
            return c
        lax.fori_loop(0, NBUCKET, zero_loop, 0)

        def hist_loop(i, mx):
            v = plsc.bitcast(row_v[pl.ds(i * L, L)], jnp.int32)
            ab = v & SIGN_MASK
            idx = ((ab >> BSHIFT) << 4) | lane
            plsc.addupdate_scatter(hist_v, [idx], ones)
            return jnp.maximum(mx, ab)
        mxv = lax.fori_loop(0, NVREG, hist_loop, zeros)
        bstart = jnp.max(mxv) >> BSHIFT

        def scan_cond(st):
            b, cum, _ = st
            return jnp.logical_and(cum < K, b >= 0)

        def scan_body(st):
            b, cum, _ = st
            c = jnp.sum(hist_v[pl.ds(b * L, L)])
            return (b - 1, cum + c, c)
        bf, cum, lastc = lax.while_loop(
            scan_cond, scan_body, (bstart, jnp.int32(0), jnp.int32(0)))
        bucket = bf + 1
        need = K - (cum - lastc)

        def collect_loop(i, cnt):
            v = plsc.bitcast(row_v[pl.ds(i * L, L)], jnp.int32)
            ab = v & SIGN_MASK
            m = jnp.logical_and((ab >> BSHIFT) == bucket, cnt < CAND_CAP - L)
            plsc.store_compressed(cand_v.at[pl.ds(cnt, L)], ab, mask=m)
            return cnt + plsc.all_reduce_population_count(m)[0]
        cnt = lax.fori_loop(0, NVREG, collect_loop, jnp.int32(0))
        cand_v[pl.ds(cnt, L)] = zeros

        nv = (cnt + L - 1) >> 4
        base = bucket << BSHIFT

        def bs_loop(_, st):
            lo, hi = st
            mid = lo + ((hi - lo + 1) >> 1)

            def count_loop(j, acc):
                pm = cand_v[pl.ds(j * L, L)] >= mid
                return acc + plsc.all_reduce_population_count(pm)[0]
            c = lax.fori_loop(0, nv, count_loop, jnp.int32(0))
            ok = c >= need
            return (jnp.where(ok, mid, lo), jnp.where(ok, hi, mid - 1))
        thr, _ = lax.fori_loop(0, 21, bs_loop,
                               (base, base + (1 << BSHIFT) - 1))

        def mask_loop(i, c):
            v = plsc.bitcast(row_v[pl.ds(i * L, L)], jnp.int32)
            keep = (v & SIGN_MASK) >= thr
            row_v[pl.ds(i * L, L)] = plsc.bitcast(
                jnp.where(keep, v, 0), jnp.float32)
            return c
        lax.fori_loop(0, NVREG, mask_loop, 0)
        pltpu.sync_copy(row_v, out_hbm.at[row])
        return carry

    lax.fori_loop(0, ROWS_PER_W, per_row, 0)


_topk_mask = functools.partial(
    pl.kernel,
    out_type=jax.ShapeDtypeStruct((ROWS, COLS), jnp.float32),
    mesh=plsc.VectorSubcoreMesh(core_axis_name="c", subcore_axis_name="s"),
    scratch_types=[
        pltpu.VMEM((COLS,), jnp.float32),
        pltpu.VMEM((NBUCKET * L,), jnp.int32),
        pltpu.VMEM((CAND_CAP + L,), jnp.int32),
    ],
)(_sc_body)


@jax.jit
def kernel(input_):
    return _topk_mask(input_)


# speedup vs baseline: 3.0307x; 3.0307x over previous
import functools

import jax
import jax.numpy as jnp
from jax import lax
from jax.experimental import pallas as pl
from jax.experimental.pallas import tpu as pltpu
from jax.experimental.pallas import tpu_sc as plsc

ROWS = 128
COLS = 32768
K = 64
L = 16
NVREG = COLS // L
BSHIFT = 21
NBUCKET = 1 << (31 - BSHIFT)
CAND_CAP = 4096
SIGN_MASK = 0x7FFFFFFF
NC = 2
NS = 16
ROWS_PER_W = ROWS // (NC * NS)


def _sc_body(in_hbm, out_hbm, row_v, hist_v, cand_v):
    wid = lax.axis_index("s") * NC + lax.axis_index("c")
    lane = lax.iota(jnp.int32, L)
    ones = jnp.ones((L,), jnp.int32)
    zeros = jnp.zeros((L,), jnp.int32)

    def per_row(r, carry):
        row = wid * ROWS_PER_W + r
        pltpu.sync_copy(in_hbm.at[row], row_v)

        def zero_loop(i, c):
            hist_v[pl.ds(i * L, L)] = zeros
            return c
        lax.fori_loop(0, NBUCKET, zero_loop, 0)

        def hist_loop(i, mx):
            v = plsc.bitcast(row_v[pl.ds(i * L, L)], jnp.int32)
            ab = v & SIGN_MASK
            idx = ((ab >> BSHIFT) << 4) | lane
            plsc.addupdate_scatter(hist_v, [idx], ones)
            return jnp.maximum(mx, ab)
        mxv = lax.fori_loop(0, NVREG, hist_loop, zeros)
        bstart = jnp.max(mxv) >> BSHIFT

        def scan_cond(st):
            b, cum, _ = st
            return jnp.logical_and(cum < K, b >= 0)

        def scan_body(st):
            b, cum, _ = st
            c = jnp.sum(hist_v[pl.ds(b * L, L)])
            return (b - 1, cum + c, c)
        bf, cum, lastc = lax.while_loop(
            scan_cond, scan_body, (bstart, jnp.int32(0), jnp.int32(0)))
        bucket = bf + 1
        need = K - (cum - lastc)

        def collect_loop(i, cnt):
            v = plsc.bitcast(row_v[pl.ds(i * L, L)], jnp.int32)
            ab = v & SIGN_MASK
            m = jnp.logical_and((ab >> BSHIFT) == bucket, cnt < CAND_CAP - L)
            plsc.store_compressed(cand_v.at[pl.ds(cnt, L)], ab, mask=m)
            return cnt + plsc.all_reduce_population_count(m)[0]
        cnt = lax.fori_loop(0, NVREG, collect_loop, jnp.int32(0))
        cand_v[pl.ds(cnt, L)] = zeros

        nv = (cnt + L - 1) >> 4
        base = bucket << BSHIFT

        def bs_loop(_, st):
            lo, hi = st
            mid = lo + ((hi - lo + 1) >> 1)

            def count_loop(j, acc):
                pm = cand_v[pl.ds(j * L, L)] >= mid
                return acc + plsc.all_reduce_population_count(pm)[0]
            c = lax.fori_loop(0, nv, count_loop, jnp.int32(0))
            ok = c >= need
            return (jnp.where(ok, mid, lo), jnp.where(ok, hi, mid - 1))
        thr, _ = lax.fori_loop(0, 21, bs_loop,
                               (base, base + (1 << BSHIFT) - 1))

        def mask_loop(i, c):
            v = plsc.bitcast(row_v[pl.ds(i * L, L)], jnp.int32)
            keep = (v & SIGN_MASK) >= thr
            row_v[pl.ds(i * L, L)] = plsc.bitcast(
                jnp.where(keep, v, 0), jnp.float32)
            return c
        lax.fori_loop(0, NVREG, mask_loop, 0)
        pltpu.sync_copy(row_v, out_hbm.at[row])
        return carry

    lax.fori_loop(0, ROWS_PER_W, per_row, 0)


_topk_mask = functools.partial(
    pl.kernel,
    out_type=jax.ShapeDtypeStruct((ROWS, COLS), jnp.float32),
    mesh=plsc.VectorSubcoreMesh(core_axis_name="c", subcore_axis_name="s"),
    scratch_types=[
        pltpu.VMEM((COLS,), jnp.float32),
        pltpu.VMEM((NBUCKET * L,), jnp.int32),
        pltpu.VMEM((CAND_CAP + L,), jnp.int32),
    ],
    compiler_params=pltpu.CompilerParams(needs_layout_passes=False),
)(_sc_body)


@jax.jit
def kernel(input_):
    return _topk_mask(input_)


# speedup vs baseline: 5.1853x; 1.7109x over previous
import functools

import jax
import jax.numpy as jnp
from jax import lax
from jax.experimental import pallas as pl
from jax.experimental.pallas import tpu as pltpu
from jax.experimental.pallas import tpu_sc as plsc

ROWS = 128
COLS = 32768
K = 64
L = 16
NVREG = COLS // L
BSHIFT = 21
NBUCKET = 1 << (31 - BSHIFT)
CAND_CAP = 4096
SIGN_MASK = 0x7FFFFFFF
NC = 2
NS = 16
ROWS_PER_W = ROWS // (NC * NS)


def _sc_body(in_hbm, out_hbm, row_v, hist_v, cand_v):
    wid = lax.axis_index("s") * NC + lax.axis_index("c")
    lane = lax.iota(jnp.int32, L)
    ones = jnp.ones((L,), jnp.int32)
    zeros = jnp.zeros((L,), jnp.int32)

    def per_row(r, carry):
        row = wid * ROWS_PER_W + r
        pltpu.sync_copy(in_hbm.at[row], row_v)

        @plsc.parallel_loop(0, NBUCKET * L, L, unroll=8)
        def zero_loop(i):
            hist_v[pl.ds(i, L)] = zeros

        def hist_loop(i, mx):
            v = plsc.bitcast(row_v[pl.ds(i, L)], jnp.int32)
            ab = v & SIGN_MASK
            idx = ((ab >> BSHIFT) << 4) | lane
            plsc.addupdate_scatter(hist_v, [idx], ones)
            return jnp.maximum(mx, ab)
        mxv = plsc.parallel_loop(0, COLS, L, unroll=8, carry=zeros)(hist_loop)
        bstart = jnp.max(mxv) >> BSHIFT

        def scan_cond(st):
            b, cum, _ = st
            return jnp.logical_and(cum < K, b >= 0)

        def scan_body(st):
            b, cum, _ = st
            c = jnp.sum(hist_v[pl.ds(b * L, L)])
            return (b - 1, cum + c, c)
        bf, cum, lastc = lax.while_loop(
            scan_cond, scan_body, (bstart, jnp.int32(0), jnp.int32(0)))
        bucket = bf + 1
        need = K - (cum - lastc)

        def collect_loop(i, cnt):
            v = plsc.bitcast(row_v[pl.ds(i, L)], jnp.int32)
            ab = v & SIGN_MASK
            m = jnp.logical_and((ab >> BSHIFT) == bucket, cnt < CAND_CAP - L)
            plsc.store_compressed(cand_v.at[pl.ds(cnt, L)], ab, mask=m)
            return cnt + plsc.all_reduce_population_count(m)[0]
        cnt = plsc.parallel_loop(0, COLS, L, unroll=8,
                                 carry=jnp.int32(0))(collect_loop)
        cand_v[pl.ds(cnt, L)] = zeros

        nv = (cnt + L - 1) >> 4
        base = bucket << BSHIFT

        def bs_loop(_, st):
            lo, hi = st
            mid = lo + ((hi - lo + 1) >> 1)

            def count_loop(j, acc):
                pm = cand_v[pl.ds(j * L, L)] >= mid
                return acc + plsc.all_reduce_population_count(pm)[0]
            c = lax.fori_loop(0, nv, count_loop, jnp.int32(0))
            ok = c >= need
            return (jnp.where(ok, mid, lo), jnp.where(ok, hi, mid - 1))
        thr, _ = lax.fori_loop(0, 21, bs_loop,
                               (base, base + (1 << BSHIFT) - 1))

        @plsc.parallel_loop(0, COLS, L, unroll=8)
        def mask_loop(i):
            v = plsc.bitcast(row_v[pl.ds(i, L)], jnp.int32)
            keep = (v & SIGN_MASK) >= thr
            row_v[pl.ds(i, L)] = plsc.bitcast(
                jnp.where(keep, v, 0), jnp.float32)
        pltpu.sync_copy(row_v, out_hbm.at[row])
        return carry

    lax.fori_loop(0, ROWS_PER_W, per_row, 0)


_topk_mask = functools.partial(
    pl.kernel,
    out_type=jax.ShapeDtypeStruct((ROWS, COLS), jnp.float32),
    mesh=plsc.VectorSubcoreMesh(core_axis_name="c", subcore_axis_name="s"),
    scratch_types=[
        pltpu.VMEM((COLS,), jnp.float32),
        pltpu.VMEM((NBUCKET * L,), jnp.int32),
        pltpu.VMEM((CAND_CAP + L,), jnp.int32),
    ],
    compiler_params=pltpu.CompilerParams(needs_layout_passes=False),
)(_sc_body)


@jax.jit
def kernel(input_):
    return _topk_mask(input_)


# speedup vs baseline: 12.4048x; 2.3923x over previous
import functools

import jax
import jax.numpy as jnp
from jax import lax
from jax.experimental import pallas as pl
from jax.experimental.pallas import tpu as pltpu
from jax.experimental.pallas import tpu_sc as plsc

ROWS = 128
COLS = 32768
K = 64
L = 16
NVREG = COLS // L
BSHIFT = 21
NBUCKET = 1 << (31 - BSHIFT)
CAND_ROWS = 256
SIGN_MASK = 0x7FFFFFFF
NC = 2
NS = 16
ROWS_PER_W = ROWS // (NC * NS)


def _sc_body(in_hbm, out_hbm, row_v, hist_v, cand_v):
    wid = lax.axis_index("s") * NC + lax.axis_index("c")
    lane = lax.iota(jnp.int32, L)
    ones = jnp.ones((L,), jnp.int32)
    zeros = jnp.zeros((L,), jnp.int32)

    def per_row(r, carry):
        row = wid * ROWS_PER_W + r
        pltpu.sync_copy(in_hbm.at[row], row_v)

        @plsc.parallel_loop(0, NBUCKET * L, L, unroll=8)
        def zero_loop(i):
            hist_v[pl.ds(i, L)] = zeros

        def hist_loop(i, mx):
            v = plsc.bitcast(row_v[pl.ds(i, L)], jnp.int32)
            ab = v & SIGN_MASK
            idx = ((ab >> BSHIFT) << 4) | lane
            plsc.addupdate_scatter(hist_v, [idx], ones)
            return jnp.maximum(mx, ab)
        mxv = plsc.parallel_loop(0, COLS, L, unroll=8, carry=zeros)(hist_loop)
        bstart = jnp.max(mxv) >> BSHIFT

        def scan_cond(st):
            b, cum, _ = st
            return jnp.logical_and(cum < K, b >= 0)

        def scan_body(st):
            b, cum, _ = st
            c = jnp.sum(hist_v[pl.ds(b * L, L)])
            return (b - 1, cum + c, c)
        bf, cum, lastc = lax.while_loop(
            scan_cond, scan_body, (bstart, jnp.int32(0), jnp.int32(0)))
        bucket = bf + 1
        need = K - (cum - lastc)

        def collect_loop(i, c_vec):
            v = plsc.bitcast(row_v[pl.ds(i, L)], jnp.int32)
            ab = v & SIGN_MASK
            m = jnp.logical_and((ab >> BSHIFT) == bucket, c_vec < CAND_ROWS)
            plsc.store_scatter(cand_v, [(c_vec << 4) | lane], ab, mask=m)
            return c_vec + jnp.where(m, 1, 0)
        c_vec = plsc.parallel_loop(0, COLS, L, unroll=8,
                                   carry=zeros)(collect_loop)
        max_c = jnp.max(c_vec)

        base = bucket << BSHIFT

        def bs_loop(_, st):
            lo, hi = st
            mid = lo + ((hi - lo + 1) >> 1)

            def count_loop(j, acc):
                cv = cand_v[pl.ds(j * L, L)]
                pm = jnp.logical_and(cv >= mid, j < c_vec)
                return acc + jnp.where(pm, 1, 0)
            c = jnp.sum(lax.fori_loop(0, max_c, count_loop, zeros))
            ok = c >= need
            return (jnp.where(ok, mid, lo), jnp.where(ok, hi, mid - 1))
        thr, _ = lax.fori_loop(0, 21, bs_loop,
                               (base, base + (1 << BSHIFT) - 1))

        def tie_loop(j, acc):
            cv = cand_v[pl.ds(j * L, L)]
            valid = j < c_vec
            ge = jnp.logical_and(cv >= thr, valid)
            eq = jnp.logical_and(cv == thr, valid)
            return (acc[0] + jnp.where(ge, 1, 0),
                    acc[1] + jnp.where(eq, 1, 0))
        gev, eqv = lax.fori_loop(0, max_c, tie_loop, (zeros, zeros))
        n_ge = (K - need) + jnp.sum(gev)
        n_eq = jnp.sum(eqv)
        t_keep = K - (n_ge - n_eq)

        @plsc.parallel_loop(0, COLS, L, unroll=8)
        def mask_loop(i):
            v = plsc.bitcast(row_v[pl.ds(i, L)], jnp.int32)
            keep = (v & SIGN_MASK) >= thr
            row_v[pl.ds(i, L)] = plsc.bitcast(
                jnp.where(keep, v, 0), jnp.float32)

        @pl.when(n_ge > K)
        def _fixup():
            def fx_cond(st):
                i, c = st
                return jnp.logical_and(i < COLS, c < n_eq)

            def fx_body(st):
                i, c = st
                v = plsc.bitcast(row_v[pl.ds(i, L)], jnp.int32)
                eqm = (v & SIGN_MASK) == thr
                rank = c + plsc.cumsum(jnp.where(eqm, 1, 0)) - 1
                drop = jnp.logical_and(eqm, rank >= t_keep)
                row_v[pl.ds(i, L)] = plsc.bitcast(
                    jnp.where(drop, 0, v), jnp.float32)
                return (i + L, c + plsc.all_reduce_population_count(eqm)[0])
            lax.while_loop(fx_cond, fx_body, (jnp.int32(0), jnp.int32(0)))
        pltpu.sync_copy(row_v, out_hbm.at[row])
        return carry

    lax.fori_loop(0, ROWS_PER_W, per_row, 0)


_topk_mask = functools.partial(
    pl.kernel,
    out_type=jax.ShapeDtypeStruct((ROWS, COLS), jnp.float32),
    mesh=plsc.VectorSubcoreMesh(core_axis_name="c", subcore_axis_name="s"),
    scratch_types=[
        pltpu.VMEM((COLS,), jnp.float32),
        pltpu.VMEM((NBUCKET * L,), jnp.int32),
        pltpu.VMEM((CAND_ROWS * L,), jnp.int32),
    ],
    compiler_params=pltpu.CompilerParams(needs_layout_passes=False),
)(_sc_body)


@jax.jit
def kernel(input_):
    return _topk_mask(input_)


# speedup vs baseline: 13.6183x; 1.0978x over previous
import functools

import jax
import jax.numpy as jnp
from jax import lax
from jax.experimental import pallas as pl
from jax.experimental.pallas import tpu as pltpu
from jax.experimental.pallas import tpu_sc as plsc

ROWS = 128
COLS = 32768
K = 64
L = 16
BSHIFT = 21
NBUCKET = 1 << (31 - BSHIFT)
CAND_ROWS = 256
SIGN_MASK = 0x7FFFFFFF
NC = 2
NS = 16
ROWS_PER_W = ROWS // (NC * NS)


def _sc_body(in_hbm, out_hbm, row_a, row_b, hist_v, cand_v,
             sem_ai, sem_bi, sem_ao, sem_bo):
    wid = lax.axis_index("s") * NC + lax.axis_index("c")
    lane = lax.iota(jnp.int32, L)
    ones = jnp.ones((L,), jnp.int32)
    zeros = jnp.zeros((L,), jnp.int32)
    base_row = wid * ROWS_PER_W
    bufs = [(row_a, sem_ai, sem_ao), (row_b, sem_bi, sem_bo)]

    def process_row(row_v, row):
        def hist_loop(i, mx):
            v = plsc.bitcast(row_v[pl.ds(i, L)], jnp.int32)
            ab = v & SIGN_MASK
            idx = ((ab >> BSHIFT) << 4) | lane
            plsc.addupdate_scatter(hist_v, [idx], ones)
            return jnp.maximum(mx, ab)
        mxv = plsc.parallel_loop(0, COLS, L, unroll=8, carry=zeros)(hist_loop)
        bstart = jnp.max(mxv) >> BSHIFT

        def scan_cond(st):
            b, cum, _ = st
            return jnp.logical_and(cum < K, b >= 0)

        def scan_body(st):
            b, cum, _ = st
            c = jnp.sum(hist_v[pl.ds(b * L, L)])
            return (b - 1, cum + c, c)
        bf, cum, lastc = lax.while_loop(
            scan_cond, scan_body, (bstart, jnp.int32(0), jnp.int32(0)))
        bucket = bf + 1
        need = K - (cum - lastc)

        def collect_loop(i, cs):
            v = plsc.bitcast(row_v[pl.ds(i, L)], jnp.int32)
            ab = v & SIGN_MASK
            m = jnp.logical_and((ab >> BSHIFT) == bucket,
                                cs < CAND_ROWS * L)
            plsc.store_scatter(cand_v, [cs], ab, mask=m)
            return cs + jnp.where(m, L, 0)
        cs = plsc.parallel_loop(0, COLS, L, unroll=8,
                                carry=lane)(collect_loop)
        c_vec = (cs - lane) >> 4
        max_c = jnp.max(c_vec)

        base = bucket << BSHIFT

        def bs_loop(_, st):
            lo, hi = st
            mid = lo + ((hi - lo + 1) >> 1)

            def count_loop(j, acc):
                cv = cand_v[pl.ds(j * L, L)]
                pm = jnp.logical_and(cv >= mid, j < c_vec)
                return acc + jnp.where(pm, 1, 0)
            c = jnp.sum(lax.fori_loop(0, max_c, count_loop, zeros))
            ok = c >= need
            return (jnp.where(ok, mid, lo), jnp.where(ok, hi, mid - 1))
        thr, _ = lax.fori_loop(0, 21, bs_loop,
                               (base, base + (1 << BSHIFT) - 1))

        def tie_loop(j, acc):
            cv = cand_v[pl.ds(j * L, L)]
            valid = j < c_vec
            ge = jnp.logical_and(cv >= thr, valid)
            eq = jnp.logical_and(cv == thr, valid)
            return (acc[0] + jnp.where(ge, 1, 0),
                    acc[1] + jnp.where(eq, 1, 0))
        gev, eqv = lax.fori_loop(0, max_c, tie_loop, (zeros, zeros))
        n_ge = (K - need) + jnp.sum(gev)
        n_eq = jnp.sum(eqv)
        t_keep = K - (n_ge - n_eq)

        @plsc.parallel_loop(0, COLS, L, unroll=8)
        def mask_loop(i):
            v = plsc.bitcast(row_v[pl.ds(i, L)], jnp.int32)
            keep = (v & SIGN_MASK) >= thr
            row_v[pl.ds(i, L)] = plsc.bitcast(
                jnp.where(keep, v, 0), jnp.float32)

        @pl.when(n_ge > K)
        def _fixup():
            def fx_cond(st):
                i, c = st
                return jnp.logical_and(i < COLS, c < n_eq)

            def fx_body(st):
                i, c = st
                v = plsc.bitcast(row_v[pl.ds(i, L)], jnp.int32)
                eqm = (v & SIGN_MASK) == thr
                rank = c + plsc.cumsum(jnp.where(eqm, 1, 0)) - 1
                drop = jnp.logical_and(eqm, rank >= t_keep)
                row_v[pl.ds(i, L)] = plsc.bitcast(
                    jnp.where(drop, 0, v), jnp.float32)
                return (i + L, c + plsc.all_reduce_population_count(eqm)[0])
            lax.while_loop(fx_cond, fx_body, (jnp.int32(0), jnp.int32(0)))

    pltpu.async_copy(in_hbm.at[base_row], row_a, sem_ai)
    for r in range(ROWS_PER_W):
        row_v, sem_i, sem_o = bufs[r % 2]

        @plsc.parallel_loop(0, NBUCKET * L, L, unroll=8)
        def zero_loop(i):
            hist_v[pl.ds(i, L)] = zeros

        pltpu.make_async_copy(in_hbm.at[base_row + r], row_v, sem_i).wait()
        if r + 1 < ROWS_PER_W:
            nbuf, nsem_i, nsem_o = bufs[(r + 1) % 2]
            if r >= 1:
                pltpu.make_async_copy(
                    nbuf, out_hbm.at[base_row + r - 1], nsem_o).wait()
            pltpu.async_copy(in_hbm.at[base_row + r + 1], nbuf, nsem_i)
        process_row(row_v, base_row + r)
        pltpu.async_copy(row_v, out_hbm.at[base_row + r], sem_o)

    pltpu.make_async_copy(
        bufs[(ROWS_PER_W - 2) % 2][0],
        out_hbm.at[base_row + ROWS_PER_W - 2],
        bufs[(ROWS_PER_W - 2) % 2][2]).wait()
    pltpu.make_async_copy(
        bufs[(ROWS_PER_W - 1) % 2][0],
        out_hbm.at[base_row + ROWS_PER_W - 1],
        bufs[(ROWS_PER_W - 1) % 2][2]).wait()


_topk_mask = functools.partial(
    pl.kernel,
    out_type=jax.ShapeDtypeStruct((ROWS, COLS), jnp.float32),
    mesh=plsc.VectorSubcoreMesh(core_axis_name="c", subcore_axis_name="s"),
    scratch_types=[
        pltpu.VMEM((COLS,), jnp.float32),
        pltpu.VMEM((COLS,), jnp.float32),
        pltpu.VMEM((NBUCKET * L,), jnp.int32),
        pltpu.VMEM((CAND_ROWS * L,), jnp.int32),
        pltpu.SemaphoreType.DMA,
        pltpu.SemaphoreType.DMA,
        pltpu.SemaphoreType.DMA,
        pltpu.SemaphoreType.DMA,
    ],
    compiler_params=pltpu.CompilerParams(needs_layout_passes=False),
)(_sc_body)


@jax.jit
def kernel(input_):
    return _topk_mask(input_)
